# baseline (speedup 1.0000x reference)
import functools

import jax
import jax.numpy as jnp
from jax import lax
from jax.experimental import pallas as pl
from jax.experimental.pallas import tpu as pltpu
from jax.experimental.pallas import tpu_sc as plsc

B = 4096
S = 200
H = 64
NW = 32
BW = B // NW
C = BW
NBUF = 4
NCHUNK = S
EPS = 1e-12
NQ = H // 16

_mesh = plsc.VectorSubcoreMesh(core_axis_name="c", subcore_axis_name="s")


@functools.partial(
    pl.kernel,
    mesh=_mesh,
    out_type=jax.ShapeDtypeStruct((S, 8, NW, 8, 128), jnp.float32),
    compiler_params=pltpu.CompilerParams(
        needs_layout_passes=False, use_tc_tiling_on_sc=False),
    scratch_types=[
        pltpu.VMEM((NBUF * C,), jnp.int32),
        pltpu.VMEM((NBUF, C, H), jnp.float32),
        pltpu.VMEM((NBUF, H, C), jnp.float32),
        pltpu.VMEM((H * 16,), jnp.float32),
        pltpu.VMEM((H * 16,), jnp.float32),
        pltpu.VMEM((S * H,), jnp.float32),
        pltpu.VMEM((H,), jnp.float32),
        pltpu.VMEM((H,), jnp.float32),
        pltpu.SemaphoreType.DMA((NBUF,)),
        pltpu.SemaphoreType.DMA((NBUF,)),
        pltpu.SemaphoreType.DMA((NBUF,)),
    ],
)
def _embed_ln(xt_hbm, wt_hbm, pos_hbm, gamma_hbm, beta_hbm, out_hbm,
              idx_v, rows_v, ot_v, ht_v, posd_v, pos_v, gam_v, bet_v,
              isem, gsem, wsem):
    cid = lax.axis_index("c")
    sid = lax.axis_index("s")
    wid = sid * 2 + cid
    b_off = wid * BW

    pltpu.sync_copy(pos_hbm, pos_v)
    pltpu.sync_copy(gamma_hbm, gam_v)
    pltpu.sync_copy(beta_hbm, bet_v)

    half = jnp.full((16,), 0.5, jnp.float32)
    three_half = jnp.full((16,), 1.5, jnp.float32)

    lanes = lax.broadcasted_iota(jnp.int32, (16,), 0)
    fifteen = jnp.full((16,), 15, jnp.int32)

    def ediag(d):
        return lax.bitwise_and(lanes + d, fifteen)

    def ht_idx(e0, d):
        return (e0 + ediag(d)) * 16 + lanes

    def fire_gather(b):
        pltpu.async_copy(
            wt_hbm.at[idx_v.at[pl.ds(b * C, C)]], rows_v.at[b], gsem.at[b])

    def compute(j, b):
        pbase = j * H
        for e0 in range(0, H, 16):
            for d in range(16):
                pd = plsc.load_gather(pos_v, [pbase + e0 + ediag(d)])
                posd_v[pl.ds(e0 * 16 + d * 16, 16)] = pd

        gq = [gam_v[pl.ds(q * 16, 16)] for q in range(NQ)]
        bq = [bet_v[pl.ds(q * 16, 16)] for q in range(NQ)]

        def group_body(gi, carry):
            b0 = gi * 16
            rows16 = b0 + lanes
            s_acc = None
            q_acc = None
            for e0 in range(0, H, 16):
                for d in range(16):
                    w = plsc.load_gather(rows_v.at[b], [rows16, e0 + ediag(d)])
                    p = posd_v[pl.ds(e0 * 16 + d * 16, 16)]
                    h = w + p
                    plsc.store_scatter(ht_v, [ht_idx(e0, d)], h)
                    s_acc = h if s_acc is None else s_acc + h
                    q_acc = h * h if q_acc is None else q_acc + h * h

            mean = s_acc * (1.0 / H)
            var = q_acc * (1.0 / H) - mean * mean
            v = var + EPS
            yi = jnp.full((16,), 0x5F3759DF, jnp.int32) - lax.shift_right_logical(
                plsc.bitcast(v, jnp.int32), jnp.full((16,), 1, jnp.int32))
            rst = plsc.bitcast(yi, jnp.float32)
            hv = half * v
            rst = rst * (three_half - hv * rst * rst)
            rst = rst * (three_half - hv * rst * rst)

            for e in range(H):
                h = ht_v[pl.ds(e * 16, 16)]
                o = (h - mean) * rst * gq[e // 16][e % 16] + bq[e // 16][e % 16]
                ot_v[b, e, pl.ds(b0, 16)] = o
            return carry

        lax.fori_loop(0, C // 16, group_body, 0)

    pltpu.sync_copy(xt_hbm.at[pl.ds(b_off, C)], idx_v.at[pl.ds(0, C)])
    fire_gather(0)
    pltpu.async_copy(xt_hbm.at[pl.ds(B + b_off, C)],
                     idx_v.at[pl.ds(C, C)], isem.at[1])

    def k_body(k, carry):
        for u in range(NBUF):
            j = k * NBUF + u
            b = u
            bn = (u + 1) % NBUF
            b2 = (u + 2) % NBUF

            @pl.when(j + 1 < NCHUNK)
            def _():
                pltpu.make_async_copy(
                    xt_hbm.at[pl.ds(0, C)],
                    idx_v.at[pl.ds(bn * C, C)], isem.at[bn]).wait()

                @pl.when(j >= NBUF - 1)
                def _():
                    for e8 in range(8):
                        pltpu.make_async_copy(
                            ot_v.at[bn, pl.ds(e8 * 8, 8)],
                            out_hbm.at[0, 0, 0], wsem.at[bn]).wait()

                fire_gather(bn)

            @pl.when(j + 2 < NCHUNK)
            def _():
                pltpu.async_copy(
                    xt_hbm.at[pl.ds((j + 2) * B + b_off, C)],
                    idx_v.at[pl.ds(b2 * C, C)], isem.at[b2])

            pltpu.make_async_copy(
                wt_hbm.at[pl.ds(0, C)], rows_v.at[b], gsem.at[b]).wait()
            compute(j, b)
            for e8 in range(8):
                pltpu.async_copy(ot_v.at[b, pl.ds(e8 * 8, 8)],
                                 out_hbm.at[j, e8, wid], wsem.at[b])
        return carry

    lax.fori_loop(0, NCHUNK // NBUF, k_body, 0)

    for j in range(NCHUNK - NBUF + 1, NCHUNK):
        b = j % NBUF
        for e8 in range(8):
            pltpu.make_async_copy(
                ot_v.at[b, pl.ds(e8 * 8, 8)],
                out_hbm.at[0, 0, 0], wsem.at[b]).wait()


def kernel(x, word_table, pos_table, gamma, beta):
    xt = jnp.swapaxes(x, 0, 1).reshape(S * B)
    out6 = _embed_ln(xt, word_table, pos_table[:S].reshape(S * H),
                     gamma, beta)
    return out6.transpose(2, 4, 0, 1, 3).reshape(B, S, H)


# speedup vs baseline: 1.0041x; 1.0041x over previous
import functools

import jax
import jax.numpy as jnp
from jax import lax
from jax.experimental import pallas as pl
from jax.experimental.pallas import tpu as pltpu
from jax.experimental.pallas import tpu_sc as plsc

B = 4096
S = 200
H = 64
NW = 32
BW = B // NW
C = BW
NBUF = 4
NCHUNK = S
EPS = 1e-12
NQ = H // 16

_mesh = plsc.VectorSubcoreMesh(core_axis_name="c", subcore_axis_name="s")


@functools.partial(
    pl.kernel,
    mesh=_mesh,
    out_type=jax.ShapeDtypeStruct((S, 8, NW, 8, 128), jnp.float32),
    compiler_params=pltpu.CompilerParams(
        needs_layout_passes=False, use_tc_tiling_on_sc=False),
    scratch_types=[
        pltpu.VMEM((NBUF * C,), jnp.int32),
        pltpu.VMEM((NBUF, C, H), jnp.float32),
        pltpu.VMEM((NBUF, H, C), jnp.float32),
        pltpu.VMEM((H * 16,), jnp.float32),
        pltpu.VMEM((H * 16,), jnp.float32),
        pltpu.VMEM((S * H,), jnp.float32),
        pltpu.VMEM((H,), jnp.float32),
        pltpu.VMEM((H,), jnp.float32),
        pltpu.SemaphoreType.DMA((NBUF,)),
        pltpu.SemaphoreType.DMA((NBUF,)),
        pltpu.SemaphoreType.DMA((NBUF,)),
    ],
)
def _embed_ln(xt_hbm, wt_hbm, pos_hbm, gamma_hbm, beta_hbm, out_hbm,
              idx_v, rows_v, ot_v, ht_v, posd_v, pos_v, gam_v, bet_v,
              isem, gsem, wsem):
    cid = lax.axis_index("c")
    sid = lax.axis_index("s")
    wid = sid * 2 + cid
    b_off = wid * BW

    pltpu.sync_copy(pos_hbm, pos_v)
    pltpu.sync_copy(gamma_hbm, gam_v)
    pltpu.sync_copy(beta_hbm, bet_v)

    half = jnp.full((16,), 0.5, jnp.float32)
    three_half = jnp.full((16,), 1.5, jnp.float32)

    lanes = lax.broadcasted_iota(jnp.int32, (16,), 0)
    fifteen = jnp.full((16,), 15, jnp.int32)

    ediag_v = [lax.bitwise_and(lanes + d, fifteen) for d in range(16)]
    htbase_v = [ediag_v[d] * 16 + lanes for d in range(16)]

    def ediag(d):
        return ediag_v[d]

    def ht_idx(e0, d):
        return htbase_v[d] + e0 * 16

    def fire_gather(b):
        pltpu.async_copy(
            wt_hbm.at[idx_v.at[pl.ds(b * C, C)]], rows_v.at[b], gsem.at[b])

    def compute(j, b):
        pbase = j * H
        for e0 in range(0, H, 16):
            for d in range(16):
                pd = plsc.load_gather(pos_v, [pbase + e0 + ediag(d)])
                posd_v[pl.ds(e0 * 16 + d * 16, 16)] = pd

        gq = [gam_v[pl.ds(q * 16, 16)] for q in range(NQ)]
        bq = [bet_v[pl.ds(q * 16, 16)] for q in range(NQ)]

        def group_body(gi, carry):
            b0 = gi * 16
            rows16 = b0 + lanes
            s_accs, q_accs = [], []
            for e0 in range(0, H, 16):
                s_acc = None
                q_acc = None
                for d in range(16):
                    w = plsc.load_gather(rows_v.at[b], [rows16, e0 + ediag(d)])
                    p = posd_v[pl.ds(e0 * 16 + d * 16, 16)]
                    h = w + p
                    plsc.store_scatter(ht_v, [ht_idx(e0, d)], h)
                    s_acc = h if s_acc is None else s_acc + h
                    q_acc = h * h if q_acc is None else q_acc + h * h
                s_accs.append(s_acc)
                q_accs.append(q_acc)
            s_acc = (s_accs[0] + s_accs[1]) + (s_accs[2] + s_accs[3])
            q_acc = (q_accs[0] + q_accs[1]) + (q_accs[2] + q_accs[3])

            mean = s_acc * (1.0 / H)
            var = q_acc * (1.0 / H) - mean * mean
            v = var + EPS
            yi = jnp.full((16,), 0x5F3759DF, jnp.int32) - lax.shift_right_logical(
                plsc.bitcast(v, jnp.int32), jnp.full((16,), 1, jnp.int32))
            rst = plsc.bitcast(yi, jnp.float32)
            hv = half * v
            rst = rst * (three_half - hv * rst * rst)
            rst = rst * (three_half - hv * rst * rst)

            for e in range(H):
                h = ht_v[pl.ds(e * 16, 16)]
                o = (h - mean) * rst * gq[e // 16][e % 16] + bq[e // 16][e % 16]
                ot_v[b, e, pl.ds(b0, 16)] = o
            return carry

        lax.fori_loop(0, C // 16, group_body, 0)

    pltpu.sync_copy(xt_hbm.at[pl.ds(b_off, C)], idx_v.at[pl.ds(0, C)])
    fire_gather(0)
    pltpu.async_copy(xt_hbm.at[pl.ds(B + b_off, C)],
                     idx_v.at[pl.ds(C, C)], isem.at[1])

    def k_body(k, carry):
        for u in range(NBUF):
            j = k * NBUF + u
            b = u
            bn = (u + 1) % NBUF
            b2 = (u + 2) % NBUF

            @pl.when(j + 1 < NCHUNK)
            def _():
                pltpu.make_async_copy(
                    xt_hbm.at[pl.ds(0, C)],
                    idx_v.at[pl.ds(bn * C, C)], isem.at[bn]).wait()

                @pl.when(j >= NBUF - 1)
                def _():
                    for e8 in range(8):
                        pltpu.make_async_copy(
                            ot_v.at[bn, pl.ds(e8 * 8, 8)],
                            out_hbm.at[0, 0, 0], wsem.at[bn]).wait()

                fire_gather(bn)

            @pl.when(j + 2 < NCHUNK)
            def _():
                pltpu.async_copy(
                    xt_hbm.at[pl.ds((j + 2) * B + b_off, C)],
                    idx_v.at[pl.ds(b2 * C, C)], isem.at[b2])

            pltpu.make_async_copy(
                wt_hbm.at[pl.ds(0, C)], rows_v.at[b], gsem.at[b]).wait()
            compute(j, b)
            for e8 in range(8):
                pltpu.async_copy(ot_v.at[b, pl.ds(e8 * 8, 8)],
                                 out_hbm.at[j, e8, wid], wsem.at[b])
        return carry

    lax.fori_loop(0, NCHUNK // NBUF, k_body, 0)

    for j in range(NCHUNK - NBUF + 1, NCHUNK):
        b = j % NBUF
        for e8 in range(8):
            pltpu.make_async_copy(
                ot_v.at[b, pl.ds(e8 * 8, 8)],
                out_hbm.at[0, 0, 0], wsem.at[b]).wait()


def kernel(x, word_table, pos_table, gamma, beta):
    xt = jnp.swapaxes(x, 0, 1).reshape(S * B)
    out6 = _embed_ln(xt, word_table, pos_table[:S].reshape(S * H),
                     gamma, beta)
    return out6.transpose(2, 4, 0, 1, 3).reshape(B, S, H)


# speedup vs baseline: 1.6812x; 1.6743x over previous
import functools

import jax
import jax.numpy as jnp
from jax import lax
from jax.experimental import pallas as pl
from jax.experimental.pallas import tpu as pltpu
from jax.experimental.pallas import tpu_sc as plsc

B = 4096
S = 200
H = 64
NW = 32
SEQ_W = B // NW
C = S
NBUF = 4
NCHUNK = SEQ_W
RU = 4
EPS = 1e-12
NQ = H // 16
IDX_SPLIT = (0, 104)

_mesh = plsc.VectorSubcoreMesh(core_axis_name="c", subcore_axis_name="s")


@functools.partial(
    pl.kernel,
    mesh=_mesh,
    out_type=jax.ShapeDtypeStruct((B * S * H // 128, 128), jnp.float32),
    compiler_params=pltpu.CompilerParams(
        needs_layout_passes=False, use_tc_tiling_on_sc=False),
    scratch_types=[
        pltpu.VMEM((NBUF, C), jnp.int32),
        pltpu.VMEM((NBUF, C, H), jnp.float32),
        pltpu.VMEM((NBUF, C * H // 128, 128), jnp.float32),
        pltpu.VMEM((S, H), jnp.float32),
        pltpu.VMEM((2, H), jnp.float32),
        pltpu.SemaphoreType.DMA((NBUF,)),
        pltpu.SemaphoreType.DMA((NBUF,)),
        pltpu.SemaphoreType.DMA((NBUF,)),
    ],
)
def _embed_ln(x_hbm, wt_hbm, pos_hbm, gamma_hbm, beta_hbm, out_hbm,
              idx_v, rows_v, orows_v, pos_v, gb_v, isem, gsem, wsem):
    cid = lax.axis_index("c")
    sid = lax.axis_index("s")
    wid = sid * 2 + cid
    seq0 = wid * SEQ_W

    pltpu.sync_copy(pos_hbm, pos_v)
    pltpu.sync_copy(gamma_hbm, gb_v.at[0])
    pltpu.sync_copy(beta_hbm, gb_v.at[1])

    gq = [gb_v[0, pl.ds(q * 16, 16)] for q in range(NQ)]
    bq = [gb_v[1, pl.ds(q * 16, 16)] for q in range(NQ)]
    half = jnp.full((16,), 0.5, jnp.float32)
    three_half = jnp.full((16,), 1.5, jnp.float32)

    def fire_gathers(b):
        for i, off in enumerate(IDX_SPLIT):
            ln = (IDX_SPLIT + (C,))[i + 1] - off
            pltpu.async_copy(
                wt_hbm.at[idx_v.at[b, pl.ds(off, ln)]],
                rows_v.at[b, pl.ds(off, ln)],
                gsem.at[b],
            )

    def compute(b):
        def row_body(rr, carry):
            hs, means, vs = [], [], []
            for ru in range(RU):
                r = rr * RU + ru
                h = []
                for q in range(NQ):
                    w = rows_v[b, r, pl.ds(q * 16, 16)]
                    p = pos_v[r, pl.ds(q * 16, 16)]
                    h.append(w + p)
                ssum = (h[0] + h[1]) + (h[2] + h[3])
                qsum = (h[0] * h[0] + h[1] * h[1]) + (h[2] * h[2] + h[3] * h[3])
                tot = jnp.full((16,), jnp.sum(ssum), jnp.float32)
                tot2 = jnp.full((16,), jnp.sum(qsum), jnp.float32)
                mean = tot * (1.0 / H)
                var = tot2 * (1.0 / H) - mean * mean
                hs.append(h)
                means.append(mean)
                vs.append(var + EPS)

            rsts = []
            for ru in range(RU):
                v = vs[ru]
                yi = jnp.full((16,), 0x5F3759DF, jnp.int32) - lax.shift_right_logical(
                    plsc.bitcast(v, jnp.int32), jnp.full((16,), 1, jnp.int32))
                rst = plsc.bitcast(yi, jnp.float32)
                hv = half * v
                rst = rst * (three_half - hv * rst * rst)
                rst = rst * (three_half - hv * rst * rst)
                rsts.append(rst)

            for ru in range(RU):
                r = rr * RU + ru
                for q in range(NQ):
                    o = (hs[ru][q] - means[ru]) * rsts[ru] * gq[q] + bq[q]
                    orows_v[b, r // 2, pl.ds((r % 2) * 64 + q * 16, 16)] = o
            return carry

        lax.fori_loop(0, C // RU, row_body, 0)

    pltpu.sync_copy(x_hbm.at[seq0], idx_v.at[0])
    fire_gathers(0)
    pltpu.async_copy(x_hbm.at[seq0 + 1], idx_v.at[1], isem.at[1])

    def k_body(k, carry):
        for u in range(NBUF):
            j = k * NBUF + u
            b = u
            bn = (u + 1) % NBUF
            b2 = (u + 2) % NBUF

            @pl.when(j + 1 < NCHUNK)
            def _():
                pltpu.make_async_copy(
                    x_hbm.at[0], idx_v.at[bn], isem.at[bn]).wait()

                @pl.when(j >= NBUF - 1)
                def _():
                    pltpu.make_async_copy(
                        orows_v.at[bn], out_hbm.at[pl.ds(0, S * H // 128)],
                        wsem.at[bn]).wait()

                fire_gathers(bn)

            @pl.when(j + 2 < NCHUNK)
            def _():
                pltpu.async_copy(
                    x_hbm.at[seq0 + j + 2], idx_v.at[b2], isem.at[b2])

            pltpu.make_async_copy(
                wt_hbm.at[pl.ds(0, C)], rows_v.at[b], gsem.at[b]).wait()
            compute(b)
            pltpu.async_copy(orows_v.at[b],
                             out_hbm.at[pl.ds((seq0 + j) * (S * H // 128), S * H // 128)],
                             wsem.at[b])
        return carry

    lax.fori_loop(0, NCHUNK // NBUF, k_body, 0)

    for j in range(NCHUNK - NBUF + 1, NCHUNK):
        b = j % NBUF
        pltpu.make_async_copy(
            orows_v.at[b], out_hbm.at[pl.ds(0, S * H // 128)],
            wsem.at[b]).wait()


def kernel(x, word_table, pos_table, gamma, beta):
    out = _embed_ln(x, word_table, pos_table[:S], gamma, beta)
    return out.reshape(B, S, H)
